# Initial kernel scaffold; baseline (speedup 1.0000x reference)
import jax
import jax.numpy as jnp
from jax.experimental import pallas as pl


def kernel(imgs, origin_shapes, Wc, Wb, Wp, Wk, pms, u_base, shp_base, exp_base):
    raise NotImplementedError("write your pallas kernel here")


# speedup vs baseline: 10.5917x; 10.5917x over previous
import functools

import jax
import jax.numpy as jnp
from jax import lax
from jax.experimental import pallas as pl
from jax.experimental.pallas import tpu as pltpu

GRID = 40
N_CELL = GRID * GRID
PATCH_DIM = 768
N_OBJS = 15
NMS_IOU = 0.4
STRIDE = 16.0
RESIZE = 640.0
N_R, N_SHP, N_EXP = 9, 50, 29


def _heads_body(p_ref, w_ref, rr_ref, s_ref, b_ref):
    X = p_ref[0]
    hm = jnp.dot(X, w_ref[...], preferred_element_type=jnp.float32)
    c0 = jax.nn.sigmoid(hm[:, 0:1])
    c1 = jax.nn.sigmoid(hm[:, 1:2])
    s0 = jnp.where(c0 > 0.5, c0, 0.0)
    s1 = jnp.where(c1 > 0.5, c1, 0.0)
    s_ref[0] = jnp.concatenate([s0, s1], axis=1)

    idx = lax.broadcasted_iota(jnp.int32, (N_CELL, 1), 0)
    cx = ((idx % GRID) * 16).astype(jnp.float32)
    cy = ((idx // GRID) * 16).astype(jnp.float32)
    d0 = hm[:, 2:3]
    d1 = hm[:, 3:4]
    d2 = hm[:, 4:5]
    d3 = hm[:, 5:6]
    rh = rr_ref[0, 0, 0]
    rw = rr_ref[0, 0, 1]
    y1 = (cy - d1) * rh
    x1 = (cx - d0) * rw
    y2 = (cy + d3) * rh
    x2 = (cx + d2) * rw
    b_ref[0] = jnp.concatenate([y1, x1, y2, x2], axis=1)


def _nms_body(s_ref, b_ref, si_ref, sb_ref, ss_ref):
    s2 = s_ref[0]
    s0 = s2[:, 0:1]
    s1 = s2[:, 1:2]
    s = jnp.concatenate([s0, s1, s0, s1], axis=1)
    bx = b_ref[0]
    y1 = bx[:, 0:1]
    x1 = bx[:, 1:2]
    y2 = bx[:, 2:3]
    x2 = bx[:, 3:4]
    area = jnp.maximum(y2 - y1, 0.0) * jnp.maximum(x2 - x1, 0.0)
    cell = lax.broadcasted_iota(jnp.int32, (N_CELL, 4), 0)
    col = lax.broadcasted_iota(jnp.int32, (N_CELL, 4), 1)
    flat = cell * 4 + col
    cell1 = lax.broadcasted_iota(jnp.int32, (N_CELL, 1), 0)
    k15 = lax.broadcasted_iota(jnp.int32, (N_OBJS, 1), 0)

    acc_i = jnp.zeros((N_OBJS, 1), jnp.int32)
    acc_s = jnp.zeros((N_OBJS, 1), jnp.float32)
    acc_b = jnp.zeros((N_OBJS, 4), jnp.float32)
    for k in range(N_OBJS):
        m = jnp.max(s)
        i = jnp.min(jnp.where(s == m, flat, 4 * N_CELL))
        ci = i // 4
        cmask = (cell1 == ci)
        by1 = jnp.max(jnp.where(cmask, y1, -jnp.inf))
        bx1 = jnp.max(jnp.where(cmask, x1, -jnp.inf))
        by2 = jnp.max(jnp.where(cmask, y2, -jnp.inf))
        bx2 = jnp.max(jnp.where(cmask, x2, -jnp.inf))
        ai = jnp.maximum(by2 - by1, 0.0) * jnp.maximum(bx2 - bx1, 0.0)
        yy1 = jnp.maximum(y1, by1)
        xx1 = jnp.maximum(x1, bx1)
        yy2 = jnp.minimum(y2, by2)
        xx2 = jnp.minimum(x2, bx2)
        inter = jnp.maximum(yy2 - yy1, 0.0) * jnp.maximum(xx2 - xx1, 0.0)
        iou = inter / (area + ai - inter + 1e-9)
        s = jnp.where((iou > NMS_IOU) | (flat == i), 0.0, s)
        valid = m > 0.0
        row = jnp.concatenate(
            [v.reshape(1, 1) for v in (by1, bx1, by2, bx2)], axis=1)
        row = jnp.where(valid, row, jnp.inf)
        acc_i = jnp.where(k15 == k, ci, acc_i)
        acc_s = jnp.where(k15 == k, jnp.where(valid, m, 0.0), acc_s)
        acc_b = jnp.where(k15 == k, row, acc_b)
    si_ref[0] = acc_i
    ss_ref[0] = acc_s
    sb_ref[0] = acc_b


def _post_body(p_ref, si_ref, wp_ref, bias_ref, bx_ref, by_ref, bz_ref,
               ux_ref, uy_ref, uz_ref, rr_ref, ln_ref):
    rows = [p_ref[0, pl.ds(si_ref[0, k, 0], 1), :] for k in range(N_OBJS)]
    X = jnp.concatenate(rows, axis=0)
    pr = jnp.dot(X, wp_ref[...], preferred_element_type=jnp.float32)
    pr = pr + bias_ref[...]
    p79 = pr[:, N_R:]
    vx = jnp.dot(p79, bx_ref[...], preferred_element_type=jnp.float32) + ux_ref[...]
    vy = jnp.dot(p79, by_ref[...], preferred_element_type=jnp.float32) + uy_ref[...]
    vz = jnp.dot(p79, bz_ref[...], preferred_element_type=jnp.float32) + uz_ref[...]
    l0 = vx * pr[:, 0:1] + vy * pr[:, 1:2] + vz * pr[:, 2:3]
    l1 = vx * pr[:, 3:4] + vy * pr[:, 4:5] + vz * pr[:, 5:6]
    rh = rr_ref[0, 0, 0]
    rw = rr_ref[0, 0, 1]
    ln_ref[0, 0] = l1 * rh
    ln_ref[0, 1] = l0 * rw


def kernel(imgs, origin_shapes, Wc, Wb, Wp, Wk, pms, u_base, shp_base, exp_base):
    B = imgs.shape[0]
    patches = imgs.reshape(B, GRID, 16, GRID, 16, 3)
    patches = patches.transpose(0, 1, 3, 2, 4, 5).reshape(B, N_CELL, PATCH_DIM)
    rr = (origin_shapes / RESIZE).reshape(B, 1, 2)

    Wcb = jnp.concatenate([Wc, Wb * STRIDE], axis=1)
    Wp2 = Wp * pms[1][None, :]
    bias = pms[0].reshape(1, N_R + N_SHP + N_EXP)
    bases = jnp.concatenate([shp_base, exp_base], axis=1)
    Bx = bases[0::3, :].T
    By = bases[1::3, :].T
    Bz = bases[2::3, :].T
    ux = u_base[0::3, 0].reshape(1, 68)
    uy = u_base[1::3, 0].reshape(1, 68)
    uz = u_base[2::3, 0].reshape(1, 68)

    score, boxes = pl.pallas_call(
        _heads_body,
        grid=(B,),
        in_specs=[
            pl.BlockSpec((1, N_CELL, PATCH_DIM), lambda b: (b, 0, 0)),
            pl.BlockSpec((PATCH_DIM, 6), lambda b: (0, 0)),
            pl.BlockSpec((1, 1, 2), lambda b: (b, 0, 0)),
        ],
        out_specs=[
            pl.BlockSpec((1, N_CELL, 2), lambda b: (b, 0, 0)),
            pl.BlockSpec((1, N_CELL, 4), lambda b: (b, 0, 0)),
        ],
        out_shape=[
            jax.ShapeDtypeStruct((B, N_CELL, 2), jnp.float32),
            jax.ShapeDtypeStruct((B, N_CELL, 4), jnp.float32),
        ],
    )(patches, Wcb, rr)

    selidx, selbox, selscore = pl.pallas_call(
        _nms_body,
        grid=(B,),
        in_specs=[
            pl.BlockSpec((1, N_CELL, 2), lambda b: (b, 0, 0)),
            pl.BlockSpec((1, N_CELL, 4), lambda b: (b, 0, 0)),
        ],
        out_specs=[
            pl.BlockSpec((1, N_OBJS, 1), lambda b: (b, 0, 0)),
            pl.BlockSpec((1, N_OBJS, 4), lambda b: (b, 0, 0)),
            pl.BlockSpec((1, N_OBJS, 1), lambda b: (b, 0, 0)),
        ],
        out_shape=[
            jax.ShapeDtypeStruct((B, N_OBJS, 1), jnp.int32),
            jax.ShapeDtypeStruct((B, N_OBJS, 4), jnp.float32),
            jax.ShapeDtypeStruct((B, N_OBJS, 1), jnp.float32),
        ],
    )(score, boxes)

    lnmk2 = pl.pallas_call(
        _post_body,
        grid=(B,),
        in_specs=[
            pl.BlockSpec((1, N_CELL, PATCH_DIM), lambda b: (b, 0, 0)),
            pl.BlockSpec((1, N_OBJS, 1), lambda b: (b, 0, 0),
                         memory_space=pltpu.SMEM),
            pl.BlockSpec((PATCH_DIM, 88), lambda b: (0, 0)),
            pl.BlockSpec((1, 88), lambda b: (0, 0)),
            pl.BlockSpec((79, 68), lambda b: (0, 0)),
            pl.BlockSpec((79, 68), lambda b: (0, 0)),
            pl.BlockSpec((79, 68), lambda b: (0, 0)),
            pl.BlockSpec((1, 68), lambda b: (0, 0)),
            pl.BlockSpec((1, 68), lambda b: (0, 0)),
            pl.BlockSpec((1, 68), lambda b: (0, 0)),
            pl.BlockSpec((1, 1, 2), lambda b: (b, 0, 0)),
        ],
        out_specs=[
            pl.BlockSpec((1, 2, N_OBJS, 68), lambda b: (b, 0, 0, 0)),
        ],
        out_shape=[
            jax.ShapeDtypeStruct((B, 2, N_OBJS, 68), jnp.float32),
        ],
    )(patches, selidx, Wp2, bias, Bx, By, Bz, ux, uy, uz, rr)[0]

    lnmk = lnmk2.transpose(0, 2, 3, 1)
    sc = selscore.reshape(B, N_OBJS)
    valid = (sc > 0.0)[:, :, None, None]
    lnmk = jnp.where(valid, lnmk, jnp.inf)
    return selbox, lnmk, sc


# speedup vs baseline: 38.7917x; 3.6625x over previous
import functools

import jax
import jax.numpy as jnp
from jax import lax
from jax.experimental import pallas as pl
from jax.experimental.pallas import tpu as pltpu
from jax.experimental.pallas import tpu_sc as plsc

GRID = 40
N_CELL = GRID * GRID
PATCH_DIM = 768
N_OBJS = 15
NMS_IOU = 0.4
STRIDE = 16.0
RESIZE = 640.0
N_R, N_SHP, N_EXP = 9, 50, 29


def _heads_body(img_ref, w_ref, rr_ref, s_ref, b_ref, pt_ref):
    Xr = img_ref[0]
    X = Xr.reshape(GRID, 16, GRID, 48).transpose(0, 2, 1, 3).reshape(
        N_CELL, PATCH_DIM)
    pt_ref[0] = X
    hm = jnp.dot(X, w_ref[...], preferred_element_type=jnp.float32)
    hmT = hm.T
    c0 = jax.nn.sigmoid(hmT[0:1])
    c1 = jax.nn.sigmoid(hmT[1:2])
    s0 = jnp.where(c0 > 0.5, c0, 0.0)
    s1 = jnp.where(c1 > 0.5, c1, 0.0)
    s_ref[0] = jnp.concatenate([s0, s1], axis=0)

    idx = lax.broadcasted_iota(jnp.int32, (1, N_CELL), 1)
    cx = ((idx % GRID) * 16).astype(jnp.float32)
    cy = ((idx // GRID) * 16).astype(jnp.float32)
    d0 = hmT[2:3]
    d1 = hmT[3:4]
    d2 = hmT[4:5]
    d3 = hmT[5:6]
    rh = rr_ref[0, 0, 0]
    rw = rr_ref[0, 0, 1]
    y1 = (cy - d1) * rh
    x1 = (cx - d0) * rw
    y2 = (cy + d3) * rh
    x2 = (cx + d2) * rw
    b_ref[0] = jnp.concatenate([y1, x1, y2, x2], axis=0)


def _nms_sc_body(score_hbm, box_hbm, patches_hbm,
                 selbox_hbm, selscore_hbm, feat_hbm,
                 sa_v, sb_v, smax_v, y1_v, x1_v, y2_v, x2_v,
                 ar_v, selb_v, sels_v, idx_v, rows_v, sem):
    wid = lax.axis_index("s") * 2 + lax.axis_index("c")

    @pl.when(wid < 8)
    def _():
        b = wid
        pltpu.sync_copy(score_hbm.at[b, 0], sa_v)
        pltpu.sync_copy(score_hbm.at[b, 1], sb_v)
        pltpu.sync_copy(box_hbm.at[b, 0], y1_v)
        pltpu.sync_copy(box_hbm.at[b, 1], x1_v)
        pltpu.sync_copy(box_hbm.at[b, 2], y2_v)
        pltpu.sync_copy(box_hbm.at[b, 3], x2_v)
        iota = lax.iota(jnp.int32, 16)
        ninf = jnp.full((16,), -jnp.inf, jnp.float32)
        zf = jnp.zeros((16,), jnp.float32)
        zi = jnp.zeros((16,), jnp.int32)
        init = (ninf, zi, zf, zf, zf, zf, zf, zf)

        def track(carry, upd, sm, cellid, y1, x1, y2, x2, sa, sb):
            rmax, ridx, ry1, rx1, ry2, rx2, rsa, rsb = carry
            return (jnp.where(upd, sm, rmax), jnp.where(upd, cellid, ridx),
                    jnp.where(upd, y1, ry1), jnp.where(upd, x1, rx1),
                    jnp.where(upd, y2, ry2), jnp.where(upd, x2, rx2),
                    jnp.where(upd, sa, rsa), jnp.where(upd, sb, rsb))

        def stage(j, carry):
            base = j * 16
            sl = pl.ds(base, 16)
            sa = sa_v[sl]
            sb = sb_v[sl]
            sm = jnp.maximum(sa, sb)
            smax_v[sl] = sm
            y1 = y1_v[sl]
            x1 = x1_v[sl]
            y2 = y2_v[sl]
            x2 = x2_v[sl]
            ar_v[sl] = jnp.maximum(y2 - y1, 0.0) * jnp.maximum(x2 - x1, 0.0)
            return track(carry, sm > carry[0], sm, base + iota,
                         y1, x1, y2, x2, sa, sb)

        best = lax.fori_loop(0, N_CELL // 16, stage, init)

        perms = [(iota + s) % 16 for s in (1, 2, 4, 8)]

        def allmax(x):
            for p in perms:
                x = jnp.maximum(x, x.at[p].get(mode="promise_in_bounds"))
            return x

        def allmin(x):
            for p in perms:
                x = jnp.minimum(x, x.at[p].get(mode="promise_in_bounds"))
            return x

        registry = []
        acc = [jnp.zeros((16,), jnp.float32) for _ in range(5)]
        acc_cell = jnp.zeros((16,), jnp.int32)
        for k in range(N_OBJS):
            rmax, ridx = best[0], best[1]
            m = allmax(rmax)
            cand = jnp.where(rmax == m, ridx, jnp.int32(N_CELL))
            ci = allmin(cand)
            win = (rmax == m) & (ridx == ci)
            by1 = allmax(jnp.where(win, best[2], -jnp.inf))
            bx1 = allmax(jnp.where(win, best[3], -jnp.inf))
            by2 = allmax(jnp.where(win, best[4], -jnp.inf))
            bx2 = allmax(jnp.where(win, best[5], -jnp.inf))
            sa_ci = allmax(jnp.where(win, best[6], -jnp.inf))
            sb_ci = allmax(jnp.where(win, best[7], -jnp.inf))
            valid = m > 0.0
            ai = jnp.maximum(by2 - by1, 0.0) * jnp.maximum(bx2 - bx1, 0.0)
            cA = jnp.full((16,), 2, jnp.int32)
            cB = jnp.full((16,), 2, jnp.int32)
            for pc, pA, pB in registry:
                mt = pc == ci
                cA = jnp.where(mt, pA, cA)
                cB = jnp.where(mt, pB, cB)
            effA = jnp.where(cA > 0, sa_ci, 0.0)
            chanA = effA == m
            cA = jnp.where(chanA, cA - 1, cA)
            cB = jnp.where(chanA, cB, cB - 1)
            registry = [(pc, jnp.where(pc == ci, cA, pA),
                         jnp.where(pc == ci, cB, pB))
                        for pc, pA, pB in registry]
            registry.append((ci, cA, cB))
            effA = jnp.where(cA > 0, sa_ci, 0.0)
            effB = jnp.where(cB > 0, sb_ci, 0.0)
            selfsupp = (ai / (ai + 1e-9)) > NMS_IOU
            new_ci_val = jnp.where(selfsupp, 0.0, jnp.maximum(effA, effB))

            km = iota == k
            vals = (jnp.where(valid, by1, jnp.inf),
                    jnp.where(valid, bx1, jnp.inf),
                    jnp.where(valid, by2, jnp.inf),
                    jnp.where(valid, bx2, jnp.inf),
                    jnp.where(valid, m, 0.0))
            acc = [jnp.where(km, v, a) for v, a in zip(vals, acc)]
            acc_cell = jnp.where(km, b * N_CELL + ci, acc_cell)

            def fuse(j, carry):
                base = j * 16
                sl = pl.ds(base, 16)
                sm = smax_v[sl]
                y1 = y1_v[sl]
                x1 = x1_v[sl]
                y2 = y2_v[sl]
                x2 = x2_v[sl]
                yy1 = jnp.maximum(y1, by1)
                xx1 = jnp.maximum(x1, bx1)
                yy2 = jnp.minimum(y2, by2)
                xx2 = jnp.minimum(x2, bx2)
                inter = jnp.maximum(yy2 - yy1, 0.0) * jnp.maximum(xx2 - xx1, 0.0)
                iou = inter / (ar_v[sl] + ai - inter + 1e-9)
                sm2 = jnp.where(iou > NMS_IOU, 0.0, sm)
                cellid = base + iota
                sm2 = jnp.where(cellid == ci, new_ci_val, sm2)
                smax_v[sl] = sm2
                return track(carry, sm2 > carry[0], sm2, cellid,
                             y1, x1, y2, x2, sa_v[sl], sb_v[sl])

            if k < N_OBJS - 1:
                best = lax.fori_loop(0, N_CELL // 16, fuse, init)

        for coord in range(4):
            selb_v[coord] = acc[coord]
        sels_v[...] = acc[4]
        idx_v[...] = acc_cell
        pltpu.sync_copy(selb_v, selbox_hbm.at[b])
        pltpu.sync_copy(sels_v, selscore_hbm.at[b])
        pltpu.async_copy(patches_hbm.at[idx_v], rows_v, sem).wait()
        pltpu.sync_copy(rows_v, feat_hbm.at[b])


def _post_body(f_ref, sbT_ref, ss_ref, wp_ref, bias_ref, bx_ref, by_ref,
               bz_ref, ux_ref, uy_ref, uz_ref, rr_ref,
               box_ref, ln_ref, sc_ref):
    X = f_ref[0]
    pr = jnp.dot(X, wp_ref[...], preferred_element_type=jnp.float32)
    pr = pr + bias_ref[...]
    p79 = pr[:, N_R:]
    vx = jnp.dot(p79, bx_ref[...], preferred_element_type=jnp.float32) + ux_ref[...]
    vy = jnp.dot(p79, by_ref[...], preferred_element_type=jnp.float32) + uy_ref[...]
    vz = jnp.dot(p79, bz_ref[...], preferred_element_type=jnp.float32) + uz_ref[...]
    l0 = vx * pr[:, 0:1] + vy * pr[:, 1:2] + vz * pr[:, 2:3]
    l1 = vx * pr[:, 3:4] + vy * pr[:, 4:5] + vz * pr[:, 5:6]
    rh = rr_ref[0, 0, 0]
    rw = rr_ref[0, 0, 1]
    sv = ss_ref[0]
    vmask = sv.T[:N_OBJS] > 0.0
    lx = jnp.where(vmask, (l1 * rh)[:N_OBJS], jnp.inf)
    ly = jnp.where(vmask, (l0 * rw)[:N_OBJS], jnp.inf)
    ln_ref[0] = jnp.stack([lx, ly], axis=-1)
    box_ref[0] = sbT_ref[0].T[:N_OBJS]
    sc_ref[0] = sv[:, :N_OBJS]


def kernel(imgs, origin_shapes, Wc, Wb, Wp, Wk, pms, u_base, shp_base, exp_base):
    B = imgs.shape[0]
    imgs_r = imgs.reshape(B, 640, 1920)
    rr = (origin_shapes / RESIZE).reshape(B, 1, 2)

    Wcb = jnp.concatenate([Wc, Wb * STRIDE], axis=1)
    Wp2 = Wp * pms[1][None, :]
    bias = pms[0].reshape(1, N_R + N_SHP + N_EXP)
    bases = jnp.concatenate([shp_base, exp_base], axis=1)
    Bx = bases[0::3, :].T
    By = bases[1::3, :].T
    Bz = bases[2::3, :].T
    ux = u_base[0::3, 0].reshape(1, 68)
    uy = u_base[1::3, 0].reshape(1, 68)
    uz = u_base[2::3, 0].reshape(1, 68)

    score, boxes, patches = pl.pallas_call(
        _heads_body,
        grid=(B,),
        in_specs=[
            pl.BlockSpec((1, 640, 1920), lambda b: (b, 0, 0)),
            pl.BlockSpec((PATCH_DIM, 6), lambda b: (0, 0)),
            pl.BlockSpec((1, 1, 2), lambda b: (b, 0, 0)),
        ],
        out_specs=[
            pl.BlockSpec((1, 2, N_CELL), lambda b: (b, 0, 0)),
            pl.BlockSpec((1, 4, N_CELL), lambda b: (b, 0, 0)),
            pl.BlockSpec((1, N_CELL, PATCH_DIM), lambda b: (b, 0, 0)),
        ],
        out_shape=[
            jax.ShapeDtypeStruct((B, 2, N_CELL), jnp.float32),
            jax.ShapeDtypeStruct((B, 4, N_CELL), jnp.float32),
            jax.ShapeDtypeStruct((B, N_CELL, PATCH_DIM), jnp.float32),
        ],
    )(imgs_r, Wcb, rr)

    nms_sc = functools.partial(
        pl.kernel,
        mesh=plsc.VectorSubcoreMesh(core_axis_name="c", subcore_axis_name="s"),
        out_type=[
            jax.ShapeDtypeStruct((B, 4, 16), jnp.float32),
            jax.ShapeDtypeStruct((B, 16), jnp.float32),
            jax.ShapeDtypeStruct((B, 16, PATCH_DIM), jnp.float32),
        ],
        scratch_types=[
            pltpu.VMEM((N_CELL,), jnp.float32),
            pltpu.VMEM((N_CELL,), jnp.float32),
            pltpu.VMEM((N_CELL,), jnp.float32),
            pltpu.VMEM((N_CELL,), jnp.float32),
            pltpu.VMEM((N_CELL,), jnp.float32),
            pltpu.VMEM((N_CELL,), jnp.float32),
            pltpu.VMEM((N_CELL,), jnp.float32),
            pltpu.VMEM((N_CELL,), jnp.float32),
            pltpu.VMEM((4, 16), jnp.float32),
            pltpu.VMEM((16,), jnp.float32),
            pltpu.VMEM((16,), jnp.int32),
            pltpu.VMEM((16, PATCH_DIM), jnp.float32),
            pltpu.SemaphoreType.DMA,
        ],
    )(_nms_sc_body)
    selboxT, selscore16, feat16 = nms_sc(
        score, boxes, patches.reshape(B * N_CELL, PATCH_DIM))

    box15, lnmk, sc15 = pl.pallas_call(
        _post_body,
        grid=(B,),
        in_specs=[
            pl.BlockSpec((1, 16, PATCH_DIM), lambda b: (b, 0, 0)),
            pl.BlockSpec((1, 4, 16), lambda b: (b, 0, 0)),
            pl.BlockSpec((1, 1, 16), lambda b: (b, 0, 0)),
            pl.BlockSpec((PATCH_DIM, 88), lambda b: (0, 0)),
            pl.BlockSpec((1, 88), lambda b: (0, 0)),
            pl.BlockSpec((79, 68), lambda b: (0, 0)),
            pl.BlockSpec((79, 68), lambda b: (0, 0)),
            pl.BlockSpec((79, 68), lambda b: (0, 0)),
            pl.BlockSpec((1, 68), lambda b: (0, 0)),
            pl.BlockSpec((1, 68), lambda b: (0, 0)),
            pl.BlockSpec((1, 68), lambda b: (0, 0)),
            pl.BlockSpec((1, 1, 2), lambda b: (b, 0, 0)),
        ],
        out_specs=[
            pl.BlockSpec((1, N_OBJS, 4), lambda b: (b, 0, 0)),
            pl.BlockSpec((1, N_OBJS, 68, 2), lambda b: (b, 0, 0, 0)),
            pl.BlockSpec((1, 1, N_OBJS), lambda b: (b, 0, 0)),
        ],
        out_shape=[
            jax.ShapeDtypeStruct((B, N_OBJS, 4), jnp.float32),
            jax.ShapeDtypeStruct((B, N_OBJS, 68, 2), jnp.float32),
            jax.ShapeDtypeStruct((B, 1, N_OBJS), jnp.float32),
        ],
    )(feat16, selboxT, selscore16.reshape(B, 1, 16),
      Wp2, bias, Bx, By, Bz, ux, uy, uz, rr)

    return box15, lnmk, sc15.reshape(B, N_OBJS)
